# baseline (speedup 1.0000x reference)
import functools

import jax
import jax.numpy as jnp
from jax.experimental import pallas as pl
from jax.experimental.pallas import tpu as pltpu


def _round_up(x, m):
    return ((x + m - 1) // m) * m


def _prep_kernel(table_ref, aug_ref, *, v_vocab):
    tv, vp = table_ref.shape
    t = table_ref[...]
    if v_vocab < vp:
        lane = jax.lax.broadcasted_iota(jnp.int32, (tv, vp), 1)
        masked = jnp.where(lane < v_vocab, t, jnp.float32(-1e30))
    else:
        masked = t
    m = jnp.max(masked, axis=-1, keepdims=True)
    p = jnp.exp(masked - m)
    if v_vocab < vp:
        p = jnp.where(lane < v_vocab, p, 0.0)
    lse = m + jnp.log(jnp.sum(p, axis=-1, keepdims=True))
    aug_ref[:, :vp] = t.astype(jnp.bfloat16)
    aug_ref[:, vp:] = jnp.broadcast_to(lse, (tv, 128)).astype(jnp.bfloat16)


def _gather_kernel(idx_ref, tgt_ref, aug_ref, logits_ref, loss_ref, *,
                   n_tokens, tile_n):
    i = pl.program_id(0)
    tn, vp = logits_ref.shape

    idx = idx_ref[...]
    tgt = tgt_ref[...]
    col = jax.lax.broadcasted_iota(jnp.int32, (tn, vp), 1)

    onehot = (col == idx).astype(jnp.bfloat16)
    aug = jnp.dot(onehot, aug_ref[...],
                  preferred_element_type=jnp.float32)
    logits = aug[:, :vp]
    logits_ref[...] = logits

    lse = aug[:, vp:vp + 1]
    picked = jnp.sum(jnp.where(col == tgt, logits, 0.0),
                     axis=-1, keepdims=True)
    row = jax.lax.broadcasted_iota(jnp.int32, (tn, 1), 0) + i * tile_n
    per_row = jnp.where(row < n_tokens, lse - picked, 0.0)
    loss_ref[...] = jnp.full(loss_ref.shape, jnp.sum(per_row), jnp.float32)


def kernel(idx, table, targets):
    idx = jnp.asarray(idx, jnp.int32)
    table = jnp.asarray(table, jnp.float32)
    B, T = idx.shape
    V = table.shape[0]
    N = B * T

    v_pad = _round_up(V, 128)
    tile_n = min(512, _round_up(N, 8))
    num_tiles = pl.cdiv(N, tile_n)
    n_pad = num_tiles * tile_n

    table_p = table if v_pad == V else jnp.pad(
        table, ((0, v_pad - V), (0, v_pad - V)))

    tile_v = 256 if v_pad % 256 == 0 else 128
    aug = pl.pallas_call(
        functools.partial(_prep_kernel, v_vocab=V),
        out_shape=jax.ShapeDtypeStruct((v_pad, v_pad + 128), jnp.bfloat16),
        grid=(v_pad // tile_v,),
        in_specs=[pl.BlockSpec((tile_v, v_pad), lambda i: (i, 0))],
        out_specs=pl.BlockSpec((tile_v, v_pad + 128), lambda i: (i, 0)),
        compiler_params=pltpu.CompilerParams(dimension_semantics=("parallel",)),
    )(table_p)

    idx_p = jnp.pad(idx.reshape(-1), (0, n_pad - N)).reshape(n_pad, 1)
    tgt_p = jnp.pad(jnp.asarray(targets, jnp.int32).reshape(-1),
                    (0, n_pad - N)).reshape(n_pad, 1)

    tok_spec = pl.BlockSpec((tile_n, 1), lambda i: (i, 0))
    logits_p, loss_part = pl.pallas_call(
        functools.partial(_gather_kernel, n_tokens=N, tile_n=tile_n),
        out_shape=(
            jax.ShapeDtypeStruct((n_pad, v_pad), jnp.float32),
            jax.ShapeDtypeStruct((num_tiles, 1, 128), jnp.float32),
        ),
        grid=(num_tiles,),
        in_specs=[
            tok_spec,
            tok_spec,
            pl.BlockSpec((v_pad, v_pad + 128), lambda i: (0, 0)),
        ],
        out_specs=(
            pl.BlockSpec((tile_n, v_pad), lambda i: (i, 0)),
            pl.BlockSpec((1, 1, 128), lambda i: (i, 0, 0)),
        ),
        compiler_params=pltpu.CompilerParams(dimension_semantics=("parallel",)),
    )(idx_p, tgt_p, aug)

    loss = loss_part[:, 0, 0].sum() / N
    logits = logits_p[:N, :V]
    return logits, loss


# speedup vs baseline: 2.2381x; 2.2381x over previous
import functools

import jax
import jax.numpy as jnp
from jax.experimental import pallas as pl
from jax.experimental.pallas import tpu as pltpu


def _round_up(x, m):
    return ((x + m - 1) // m) * m


def _lse_kernel(table_ref, lse_ref, *, v_vocab):
    tv, vp = table_ref.shape
    t = table_ref[...]
    if v_vocab < vp:
        lane = jax.lax.broadcasted_iota(jnp.int32, (tv, vp), 1)
        t = jnp.where(lane < v_vocab, t, jnp.float32(-1e30))
    m = jnp.max(t, axis=-1, keepdims=True)
    p = jnp.exp(t - m)
    if v_vocab < vp:
        p = jnp.where(lane < v_vocab, p, 0.0)
    lse = m + jnp.log(jnp.sum(p, axis=-1, keepdims=True))
    lse_ref[...] = jnp.broadcast_to(lse, (tv, 128))


def _gather_kernel(idx_smem, tgt_ref, tg_ref, logits_ref, loss_ref,
                   scratch_ref, *, n_tokens, tile_n, n_chunk, stride):
    i = pl.program_id(0)
    tn, vp = logits_ref.shape
    n_aug = tg_ref.shape[1]

    for mi in range(tile_n):
        slab = tg_ref[idx_smem[mi]]
        scratch_ref[pl.ds(mi, n_aug, stride), :] = slab

    planes = [scratch_ref[pl.ds(c * stride, tn), :] for c in range(n_chunk)]
    logits = jnp.concatenate(planes, axis=1)
    logits_ref[...] = logits

    lse = scratch_ref[pl.ds(n_chunk * stride, tn), 0:1]
    tgt = tgt_ref[...]
    col = jax.lax.broadcasted_iota(jnp.int32, (tn, vp), 1)
    picked = jnp.sum(jnp.where(col == tgt, logits, 0.0),
                     axis=-1, keepdims=True)
    row = jax.lax.broadcasted_iota(jnp.int32, (tn, 1), 0) + i * tile_n
    per_row = jnp.where(row < n_tokens, lse - picked, 0.0)
    loss_ref[...] = jnp.full(loss_ref.shape, jnp.sum(per_row), jnp.float32)


def kernel(idx, table, targets):
    idx = jnp.asarray(idx, jnp.int32)
    table = jnp.asarray(table, jnp.float32)
    B, T = idx.shape
    V = table.shape[0]
    N = B * T

    v_pad = _round_up(V, 128)
    n_chunk = v_pad // 128
    n_aug = _round_up(n_chunk + 1, 8)
    tile_n = min(512, _round_up(N, 8))
    num_tiles = pl.cdiv(N, tile_n)
    n_pad = num_tiles * tile_n
    stride = tile_n + 8
    scratch_rows = stride * (n_aug - 1) + tile_n

    table_p = table if v_pad == V else jnp.pad(
        table, ((0, v_pad - V), (0, v_pad - V)))

    tile_v = 256 if v_pad % 256 == 0 else 128
    lse = pl.pallas_call(
        functools.partial(_lse_kernel, v_vocab=V),
        out_shape=jax.ShapeDtypeStruct((v_pad, 128), jnp.float32),
        grid=(v_pad // tile_v,),
        in_specs=[pl.BlockSpec((tile_v, v_pad), lambda i: (i, 0))],
        out_specs=pl.BlockSpec((tile_v, 128), lambda i: (i, 0)),
        compiler_params=pltpu.CompilerParams(dimension_semantics=("parallel",)),
    )(table_p)

    tg = jnp.concatenate(
        [table_p.reshape(v_pad, n_chunk, 128),
         lse.reshape(v_pad, 1, 128),
         jnp.zeros((v_pad, n_aug - n_chunk - 1, 128), jnp.float32)], axis=1)

    idx_p = jnp.pad(idx.reshape(-1), (0, n_pad - N))
    tgt_p = jnp.pad(jnp.asarray(targets, jnp.int32).reshape(-1),
                    (0, n_pad - N)).reshape(n_pad, 1)

    logits_p, loss_part = pl.pallas_call(
        functools.partial(_gather_kernel, n_tokens=N, tile_n=tile_n,
                          n_chunk=n_chunk, stride=stride),
        out_shape=(
            jax.ShapeDtypeStruct((n_pad, v_pad), jnp.float32),
            jax.ShapeDtypeStruct((num_tiles, 1, 128), jnp.float32),
        ),
        grid=(num_tiles,),
        in_specs=[
            pl.BlockSpec((tile_n,), lambda i: (i,),
                         memory_space=pltpu.SMEM),
            pl.BlockSpec((tile_n, 1), lambda i: (i, 0)),
            pl.BlockSpec((v_pad, n_aug, 128), lambda i: (0, 0, 0)),
        ],
        out_specs=(
            pl.BlockSpec((tile_n, v_pad), lambda i: (i, 0)),
            pl.BlockSpec((1, 1, 128), lambda i: (i, 0, 0)),
        ),
        scratch_shapes=[pltpu.VMEM((scratch_rows, 128), jnp.float32)],
        compiler_params=pltpu.CompilerParams(dimension_semantics=("parallel",)),
    )(idx_p, tgt_p, tg)

    loss = loss_part[:, 0, 0].sum() / N
    logits = logits_p[:N, :V]
    return logits, loss


# speedup vs baseline: 3.0925x; 1.3818x over previous
import functools

import jax
import jax.numpy as jnp
from jax.experimental import pallas as pl
from jax.experimental.pallas import tpu as pltpu


def _round_up(x, m):
    return ((x + m - 1) // m) * m


def _prep_kernel(table_ref, tg_ref, *, v_vocab, n_aug):
    tv, vp = table_ref.shape
    n_chunk = vp // 128
    t = table_ref[...]
    if v_vocab < vp:
        lane = jax.lax.broadcasted_iota(jnp.int32, (tv, vp), 1)
        t = jnp.where(lane < v_vocab, t, jnp.float32(-1e30))
    m = jnp.max(t, axis=-1, keepdims=True)
    p = jnp.exp(t - m)
    if v_vocab < vp:
        p = jnp.where(lane < v_vocab, p, 0.0)
        t = table_ref[...]
    lse = m + jnp.log(jnp.sum(p, axis=-1, keepdims=True))
    for c in range(n_chunk):
        tg_ref[pl.ds(c, tv, n_aug), :] = t[:, c * 128:(c + 1) * 128]
    tg_ref[pl.ds(n_chunk, tv, n_aug), :] = jnp.broadcast_to(lse, (tv, 128))
    for c in range(n_chunk + 1, n_aug):
        tg_ref[pl.ds(c, tv, n_aug), :] = jnp.zeros((tv, 128), jnp.float32)


def _gather_kernel(idx_smem, tgt_ref, tg_ref, logits_ref, loss_ref,
                   scratch_ref, *, n_tokens, tile_n, n_aug, stride):
    i = pl.program_id(0)
    tn, vp = logits_ref.shape
    n_chunk = vp // 128

    for mi in range(tile_n):
        base = pl.multiple_of(idx_smem[mi] * n_aug, 8)
        slab = tg_ref[pl.ds(base, n_aug), :]
        scratch_ref[pl.ds(mi, n_aug, stride), :] = slab

    planes = [scratch_ref[pl.ds(c * stride, tn), :] for c in range(n_chunk)]
    logits = jnp.concatenate(planes, axis=1)
    logits_ref[...] = logits

    lse = scratch_ref[pl.ds(n_chunk * stride, tn), 0:1]
    tgt = tgt_ref[...]
    col = jax.lax.broadcasted_iota(jnp.int32, (tn, vp), 1)
    picked = jnp.sum(jnp.where(col == tgt, logits, 0.0),
                     axis=-1, keepdims=True)
    row = jax.lax.broadcasted_iota(jnp.int32, (tn, 1), 0) + i * tile_n
    per_row = jnp.where(row < n_tokens, lse - picked, 0.0)
    loss_ref[...] = jnp.full(loss_ref.shape, jnp.sum(per_row), jnp.float32)


def kernel(idx, table, targets):
    idx = jnp.asarray(idx, jnp.int32)
    table = jnp.asarray(table, jnp.float32)
    B, T = idx.shape
    V = table.shape[0]
    N = B * T

    v_pad = _round_up(V, 128)
    n_chunk = v_pad // 128
    n_aug = _round_up(n_chunk + 1, 8)
    tile_n = min(512, _round_up(N, 8))
    num_tiles = pl.cdiv(N, tile_n)
    n_pad = num_tiles * tile_n
    stride = tile_n + 8
    scratch_rows = stride * (n_aug - 1) + tile_n

    table_p = table if v_pad == V else jnp.pad(
        table, ((0, v_pad - V), (0, v_pad - V)))

    tile_v = 256 if v_pad % 256 == 0 else 128
    tg = pl.pallas_call(
        functools.partial(_prep_kernel, v_vocab=V, n_aug=n_aug),
        out_shape=jax.ShapeDtypeStruct((v_pad * n_aug, 128), jnp.float32),
        grid=(v_pad // tile_v,),
        in_specs=[pl.BlockSpec((tile_v, v_pad), lambda i: (i, 0))],
        out_specs=pl.BlockSpec((tile_v * n_aug, 128), lambda i: (i, 0)),
        compiler_params=pltpu.CompilerParams(dimension_semantics=("parallel",)),
    )(table_p)

    idx_p = jnp.pad(idx.reshape(-1), (0, n_pad - N))
    tgt_p = jnp.pad(jnp.asarray(targets, jnp.int32).reshape(-1),
                    (0, n_pad - N)).reshape(n_pad, 1)

    logits_p, loss_part = pl.pallas_call(
        functools.partial(_gather_kernel, n_tokens=N, tile_n=tile_n,
                          n_aug=n_aug, stride=stride),
        out_shape=(
            jax.ShapeDtypeStruct((n_pad, v_pad), jnp.float32),
            jax.ShapeDtypeStruct((num_tiles, 1, 128), jnp.float32),
        ),
        grid=(num_tiles,),
        in_specs=[
            pl.BlockSpec((tile_n,), lambda i: (i,),
                         memory_space=pltpu.SMEM),
            pl.BlockSpec((tile_n, 1), lambda i: (i, 0)),
            pl.BlockSpec((v_pad * n_aug, 128), lambda i: (0, 0)),
        ],
        out_specs=(
            pl.BlockSpec((tile_n, v_pad), lambda i: (i, 0)),
            pl.BlockSpec((1, 1, 128), lambda i: (i, 0, 0)),
        ),
        scratch_shapes=[pltpu.VMEM((scratch_rows, 128), jnp.float32)],
        compiler_params=pltpu.CompilerParams(dimension_semantics=("parallel",)),
    )(idx_p, tgt_p, tg)

    loss = loss_part[:, 0, 0].sum() / N
    logits = logits_p[:N, :V]
    return logits, loss


# speedup vs baseline: 3.1899x; 1.0315x over previous
import functools

import jax
import jax.numpy as jnp
from jax.experimental import pallas as pl
from jax.experimental.pallas import tpu as pltpu


def _round_up(x, m):
    return ((x + m - 1) // m) * m


def _prep_kernel(table_ref, tg_ref, *, v_vocab, n_aug):
    tv, vp = table_ref.shape
    n_chunk = vp // 128
    t = table_ref[...]
    if v_vocab < vp:
        lane = jax.lax.broadcasted_iota(jnp.int32, (tv, vp), 1)
        t = jnp.where(lane < v_vocab, t, jnp.float32(-1e30))
    m = jnp.max(t, axis=-1, keepdims=True)
    p = jnp.exp(t - m)
    if v_vocab < vp:
        p = jnp.where(lane < v_vocab, p, 0.0)
        t = table_ref[...]
    lse = m + jnp.log(jnp.sum(p, axis=-1, keepdims=True))
    for c in range(n_chunk):
        tg_ref[pl.ds(c, tv, n_aug), :] = t[:, c * 128:(c + 1) * 128]
    tg_ref[pl.ds(n_chunk, tv, n_aug), :] = jnp.broadcast_to(lse, (tv, 128))


def _gather_kernel(idx_smem, tgt_ref, tg_ref, logits_ref, loss_ref,
                   scratch_ref, *, n_tokens, tile_n, sub_n, n_aug, stride):
    i = pl.program_id(0)
    tn, vp = logits_ref.shape
    n_chunk = vp // 128
    n_slab = n_chunk + 1

    partial = jnp.zeros((), jnp.float32)
    col = jax.lax.broadcasted_iota(jnp.int32, (sub_n, vp), 1)
    for h in range(tile_n // sub_n):
        for j in range(sub_n):
            base = pl.multiple_of(idx_smem[h * sub_n + j] * n_aug, 8)
            slab = tg_ref[pl.ds(base, n_slab), :]
            scratch_ref[pl.ds(j, n_slab, stride), :] = slab

        planes = [scratch_ref[pl.ds(c * stride, sub_n), :]
                  for c in range(n_chunk)]
        logits = jnp.concatenate(planes, axis=1)
        logits_ref[h * sub_n:(h + 1) * sub_n, :] = logits

        lse = scratch_ref[pl.ds(n_chunk * stride, sub_n), 0:1]
        tgt = tgt_ref[h * sub_n:(h + 1) * sub_n, :]
        picked = jnp.sum(jnp.where(col == tgt, logits, 0.0),
                         axis=-1, keepdims=True)
        row = (jax.lax.broadcasted_iota(jnp.int32, (sub_n, 1), 0)
               + (i * tile_n + h * sub_n))
        per_row = jnp.where(row < n_tokens, lse - picked, 0.0)
        partial = partial + jnp.sum(per_row)
    loss_ref[...] = jnp.full(loss_ref.shape, partial, jnp.float32)


def kernel(idx, table, targets):
    idx = jnp.asarray(idx, jnp.int32)
    table = jnp.asarray(table, jnp.float32)
    B, T = idx.shape
    V = table.shape[0]
    N = B * T

    v_pad = _round_up(V, 128)
    n_chunk = v_pad // 128
    n_aug = _round_up(n_chunk + 1, 8)
    tile_n = min(1024, _round_up(N, 8))
    sub_n = 512 if tile_n % 512 == 0 else tile_n
    num_tiles = pl.cdiv(N, tile_n)
    n_pad = num_tiles * tile_n
    stride = sub_n + 8
    scratch_rows = stride * n_chunk + sub_n

    table_p = table if v_pad == V else jnp.pad(
        table, ((0, v_pad - V), (0, v_pad - V)))

    tile_v = 256 if v_pad % 256 == 0 else 128
    tg = pl.pallas_call(
        functools.partial(_prep_kernel, v_vocab=V, n_aug=n_aug),
        out_shape=jax.ShapeDtypeStruct((v_pad * n_aug, 128), jnp.float32),
        grid=(v_pad // tile_v,),
        in_specs=[pl.BlockSpec((tile_v, v_pad), lambda i: (i, 0))],
        out_specs=pl.BlockSpec((tile_v * n_aug, 128), lambda i: (i, 0)),
        compiler_params=pltpu.CompilerParams(dimension_semantics=("parallel",)),
    )(table_p)

    idx_p = jnp.pad(idx.reshape(-1), (0, n_pad - N))
    tgt_p = jnp.pad(jnp.asarray(targets, jnp.int32).reshape(-1),
                    (0, n_pad - N)).reshape(n_pad, 1)

    logits_p, loss_part = pl.pallas_call(
        functools.partial(_gather_kernel, n_tokens=N, tile_n=tile_n,
                          sub_n=sub_n, n_aug=n_aug, stride=stride),
        out_shape=(
            jax.ShapeDtypeStruct((n_pad, v_pad), jnp.float32),
            jax.ShapeDtypeStruct((num_tiles, 1, 128), jnp.float32),
        ),
        grid=(num_tiles,),
        in_specs=[
            pl.BlockSpec((tile_n,), lambda i: (i,),
                         memory_space=pltpu.SMEM),
            pl.BlockSpec((tile_n, 1), lambda i: (i, 0)),
            pl.BlockSpec((v_pad * n_aug, 128), lambda i: (0, 0)),
        ],
        out_specs=(
            pl.BlockSpec((tile_n, v_pad), lambda i: (i, 0)),
            pl.BlockSpec((1, 1, 128), lambda i: (i, 0, 0)),
        ),
        scratch_shapes=[pltpu.VMEM((scratch_rows, 128), jnp.float32)],
        compiler_params=pltpu.CompilerParams(dimension_semantics=("parallel",)),
    )(idx_p, tgt_p, tg)

    loss = loss_part[:, 0, 0].sum() / N
    logits = logits_p[:N, :V]
    return logits, loss


# speedup vs baseline: 3.2861x; 1.0302x over previous
import functools

import jax
import jax.numpy as jnp
from jax.experimental import pallas as pl
from jax.experimental.pallas import tpu as pltpu


def _round_up(x, m):
    return ((x + m - 1) // m) * m


def _fused_kernel(idx_smem, tgt_ref, table_ref, logits_ref, loss_ref,
                  tg_ref, scratch_ref, *, n_tokens, v_vocab, tile_n, tile_v,
                  n_prep, n_aug, stride):
    i = pl.program_id(0)
    tn, vp = logits_ref.shape
    n_chunk = vp // 128
    n_slab = n_chunk + 1

    @pl.when(i < n_prep)
    def _prep():
        t = table_ref[...]
        if v_vocab < vp:
            lane = jax.lax.broadcasted_iota(jnp.int32, (tile_v, vp), 1)
            t = jnp.where(lane < v_vocab, t, jnp.float32(-1e30))
        m = jnp.max(t, axis=-1, keepdims=True)
        p = jnp.exp(t - m)
        if v_vocab < vp:
            p = jnp.where(lane < v_vocab, p, 0.0)
            t = table_ref[...]
        lse = m + jnp.log(jnp.sum(p, axis=-1, keepdims=True))
        base = i * tile_v * n_aug
        for c in range(n_chunk):
            tg_ref[pl.ds(base + c, tile_v, n_aug), :] = \
                t[:, c * 128:(c + 1) * 128]
        tg_ref[pl.ds(base + n_chunk, tile_v, n_aug), :] = \
            jnp.broadcast_to(lse, (tile_v, 128))

    @pl.when(i >= n_prep)
    def _gather():
        for mi in range(tile_n):
            base = pl.multiple_of(idx_smem[mi] * n_aug, 8)
            slab = tg_ref[pl.ds(base, n_slab), :]
            scratch_ref[pl.ds(mi, n_slab, stride), :] = slab

        planes = [scratch_ref[pl.ds(c * stride, tn), :]
                  for c in range(n_chunk)]
        logits = jnp.concatenate(planes, axis=1)
        logits_ref[...] = logits

        lse = scratch_ref[pl.ds(n_chunk * stride, tn), 0:1]
        tgt = tgt_ref[...]
        col = jax.lax.broadcasted_iota(jnp.int32, (tn, vp), 1)
        picked = jnp.sum(jnp.where(col == tgt, logits, 0.0),
                         axis=-1, keepdims=True)
        row = (jax.lax.broadcasted_iota(jnp.int32, (tn, 1), 0)
               + (i - n_prep) * tile_n)
        per_row = jnp.where(row < n_tokens, lse - picked, 0.0)
        loss_ref[...] = jnp.full(loss_ref.shape, jnp.sum(per_row),
                                 jnp.float32)


def kernel(idx, table, targets):
    idx = jnp.asarray(idx, jnp.int32)
    table = jnp.asarray(table, jnp.float32)
    B, T = idx.shape
    V = table.shape[0]
    N = B * T

    v_pad = _round_up(V, 128)
    n_chunk = v_pad // 128
    n_aug = _round_up(n_chunk + 1, 8)
    tile_n = min(512, _round_up(N, 8))
    num_tiles = pl.cdiv(N, tile_n)
    n_pad = num_tiles * tile_n
    stride = tile_n + 8
    scratch_rows = stride * n_chunk + tile_n
    tile_v = 256 if v_pad % 256 == 0 else 128
    n_prep = v_pad // tile_v

    table_p = table if v_pad == V else jnp.pad(
        table, ((0, v_pad - V), (0, v_pad - V)))

    idx_p = jnp.pad(idx.reshape(-1), (0, n_pad - N))
    tgt_p = jnp.pad(jnp.asarray(targets, jnp.int32).reshape(-1),
                    (0, n_pad - N)).reshape(n_pad, 1)

    gi = lambda i: jnp.maximum(i - n_prep, 0)
    logits_p, loss_part = pl.pallas_call(
        functools.partial(_fused_kernel, n_tokens=N, v_vocab=V,
                          tile_n=tile_n, tile_v=tile_v, n_prep=n_prep,
                          n_aug=n_aug, stride=stride),
        out_shape=(
            jax.ShapeDtypeStruct((n_pad, v_pad), jnp.float32),
            jax.ShapeDtypeStruct((num_tiles, 1, 128), jnp.float32),
        ),
        grid=(n_prep + num_tiles,),
        in_specs=[
            pl.BlockSpec((tile_n,), lambda i: (gi(i),),
                         memory_space=pltpu.SMEM),
            pl.BlockSpec((tile_n, 1), lambda i: (gi(i), 0)),
            pl.BlockSpec((tile_v, v_pad),
                         lambda i: (jnp.minimum(i, n_prep - 1), 0)),
        ],
        out_specs=(
            pl.BlockSpec((tile_n, v_pad), lambda i: (gi(i), 0)),
            pl.BlockSpec((1, 1, 128), lambda i: (gi(i), 0, 0)),
        ),
        scratch_shapes=[
            pltpu.VMEM((v_pad * n_aug, 128), jnp.float32),
            pltpu.VMEM((scratch_rows, 128), jnp.float32),
        ],
        compiler_params=pltpu.CompilerParams(
            dimension_semantics=("arbitrary",)),
    )(idx_p, tgt_p, table_p)

    loss = loss_part[:, 0, 0].sum() / N
    logits = logits_p[:N, :V]
    return logits, loss
